# baseline (speedup 1.0000x reference)
import functools

import jax
import jax.numpy as jnp
from jax.experimental import pallas as pl
from jax.experimental.pallas import tpu as pltpu

_M = 0.25
_A = 64.0 * 1.4426950408889634
_LN2 = 0.6931471805599453
_NEG_INF = float("-inf")


def _loss_kernel(label_ref, x_ref, out_ref, m_ref, s_ref, g_ref, *,
                 n_cols, block_cols):
    k = pl.program_id(0)
    nk = pl.num_programs(0)
    sqrt_a = _A ** 0.5

    @pl.when(k == 0)
    def _init():
        m_ref[...] = jnp.full(m_ref.shape, _NEG_INF, m_ref.dtype)
        s_ref[...] = jnp.zeros(s_ref.shape, s_ref.dtype)
        g_ref[...] = jnp.zeros(g_ref.shape, g_ref.dtype)

    def _accum(mask_invalid):
        x = x_ref[...]
        labloc = label_ref[...] - k * block_cols
        col = jax.lax.broadcasted_iota(jnp.int32, x.shape, 1)
        is_lab = col == labloc
        if mask_invalid:
            drop = is_lab | (col >= n_cols - k * block_cols)
        else:
            drop = is_lab
        ones = jnp.ones((x.shape[1], 1), jnp.float32)
        y = x * sqrt_a
        z = jnp.where(x > -_M, y * y, _A / 16.0)
        z = jnp.where(drop, _NEG_INF, z)
        g_ref[...] += jax.lax.dot(jnp.where(is_lab, x, 0.0), ones,
                                  preferred_element_type=jnp.float32)
        bm = jnp.max(z, axis=1, keepdims=True)
        mz_old = m_ref[...]
        mz_new = jnp.maximum(mz_old, bm)
        e = jnp.exp2(z - mz_new)
        s_ref[...] = s_ref[...] * jnp.exp2(mz_old - mz_new) + (
            jax.lax.dot(e, ones, preferred_element_type=jnp.float32))
        m_ref[...] = mz_new

    @pl.when(k < nk - 1)
    def _main():
        _accum(False)

    @pl.when(k == nk - 1)
    def _last():
        _accum(True)

        g = g_ref[...]
        spec2 = jnp.where(g < 1.0 + _M,
                          (2.0 * _A) * g - g * g * _A - 0.9375 * _A, 0.0)
        m2 = m_ref[...] - (_A / 16.0)
        s_true = s_ref[...] + jnp.exp2(spec2 - m2)
        loss = (m2 + jnp.log2(s_true) - spec2) * _LN2
        out_ref[0, 0] = jnp.sum(loss) / loss.shape[0]


def kernel(inp, label):
    b, c = inp.shape
    block_cols = 2048
    nk = pl.cdiv(c, block_cols)
    lab2 = label.astype(jnp.int32).reshape(b, 1)
    out = pl.pallas_call(
        functools.partial(_loss_kernel, n_cols=c, block_cols=block_cols),
        grid=(nk,),
        in_specs=[
            pl.BlockSpec((b, 1), lambda k: (0, 0)),
            pl.BlockSpec((b, block_cols), lambda k: (0, k)),
        ],
        out_specs=pl.BlockSpec(memory_space=pltpu.SMEM),
        out_shape=jax.ShapeDtypeStruct((1, 1), jnp.float32),
        scratch_shapes=[
            pltpu.VMEM((b, 1), jnp.float32),
            pltpu.VMEM((b, 1), jnp.float32),
            pltpu.VMEM((b, 1), jnp.float32),
        ],
    )(lab2, inp)
    return out[0, 0]


# speedup vs baseline: 1.0227x; 1.0227x over previous
import functools

import jax
import jax.numpy as jnp
from jax.experimental import pallas as pl
from jax.experimental.pallas import tpu as pltpu

_M = 0.25
_A = 64.0 * 1.4426950408889634
_LN2 = 0.6931471805599453
_NEG_INF = float("-inf")


def _loss_kernel(label_ref, x_ref, out_ref, m_ref, s_ref, g_ref, *,
                 n_cols, block_cols):
    k = pl.program_id(0)
    nk = pl.num_programs(0)

    @pl.when(k == 0)
    def _init():
        m_ref[...] = jnp.full(m_ref.shape, _NEG_INF, m_ref.dtype)
        s_ref[...] = jnp.zeros(s_ref.shape, s_ref.dtype)
        g_ref[...] = jnp.zeros(g_ref.shape, g_ref.dtype)

    def _accum(mask_invalid):
        x = x_ref[...]
        labloc = label_ref[...] - k * block_cols
        col = jax.lax.broadcasted_iota(jnp.int32, x.shape, 1)
        is_lab = col == labloc
        if mask_invalid:
            drop = is_lab | (col >= n_cols - k * block_cols)
        else:
            drop = is_lab
        ones = jnp.ones((x.shape[1], 1), jnp.float32)
        z = jnp.where(x > -_M, x * x, 1.0 / 16.0)
        z = jnp.where(drop, _NEG_INF, z)
        g_ref[...] += jax.lax.dot(jnp.where(is_lab, x, 0.0), ones,
                                  preferred_element_type=jnp.float32)
        bm = jnp.max(z, axis=1, keepdims=True)
        mz_old = m_ref[...]
        mz_new = jnp.maximum(mz_old, bm)
        e = jnp.exp2(_A * (z - mz_new))
        s_ref[...] = s_ref[...] * jnp.exp2(_A * (mz_old - mz_new)) + (
            jax.lax.dot(e, ones, preferred_element_type=jnp.float32))
        m_ref[...] = mz_new

    @pl.when(k < nk - 1)
    def _main():
        _accum(False)

    @pl.when(k == nk - 1)
    def _last():
        _accum(True)

        g = g_ref[...]
        spec2 = jnp.where(g < 1.0 + _M,
                          (2.0 * _A) * g - g * g * _A - 0.9375 * _A, 0.0)
        m2 = _A * m_ref[...] - (_A / 16.0)
        s_true = s_ref[...] + jnp.exp2(spec2 - m2)
        loss = (m2 + jnp.log2(s_true) - spec2) * _LN2
        out_ref[0, 0] = jnp.sum(loss) / loss.shape[0]


def kernel(inp, label):
    b, c = inp.shape
    block_cols = 2048
    nk = pl.cdiv(c, block_cols)
    lab2 = label.astype(jnp.int32).reshape(b, 1)
    out = pl.pallas_call(
        functools.partial(_loss_kernel, n_cols=c, block_cols=block_cols),
        grid=(nk,),
        in_specs=[
            pl.BlockSpec((b, 1), lambda k: (0, 0)),
            pl.BlockSpec((b, block_cols), lambda k: (0, k)),
        ],
        out_specs=pl.BlockSpec(memory_space=pltpu.SMEM),
        out_shape=jax.ShapeDtypeStruct((1, 1), jnp.float32),
        scratch_shapes=[
            pltpu.VMEM((b, 1), jnp.float32),
            pltpu.VMEM((b, 1), jnp.float32),
            pltpu.VMEM((b, 1), jnp.float32),
        ],
    )(lab2, inp)
    return out[0, 0]


# speedup vs baseline: 1.0438x; 1.0207x over previous
import functools

import jax
import jax.numpy as jnp
from jax.experimental import pallas as pl
from jax.experimental.pallas import tpu as pltpu

_M = 0.25
_A = 64.0 * 1.4426950408889634
_LN2 = 0.6931471805599453
_NEG_INF = float("-inf")


def _loss_kernel(label_ref, x_ref, out_ref, m_ref, s_ref, g_ref, *,
                 n_cols, block_cols):
    k = pl.program_id(0)
    nk = pl.num_programs(0)

    @pl.when(k == 0)
    def _init():
        m_ref[...] = jnp.full(m_ref.shape, _NEG_INF, m_ref.dtype)
        s_ref[...] = jnp.zeros(s_ref.shape, s_ref.dtype)
        g_ref[...] = jnp.zeros(g_ref.shape, g_ref.dtype)

    def _accum(mask_invalid):
        x = x_ref[...]
        labloc = label_ref[...] - k * block_cols
        col = jax.lax.broadcasted_iota(jnp.int32, x.shape, 1)
        is_lab = col == labloc
        if mask_invalid:
            drop = is_lab | (col >= n_cols - k * block_cols)
        else:
            drop = is_lab
        ones = jnp.ones((x.shape[1], 1), jnp.float32)
        u = jnp.maximum(x, -_M)
        z = jnp.where(drop, _NEG_INF, u * u)
        g_ref[...] += jax.lax.dot(jnp.where(is_lab, x, 0.0), ones,
                                  preferred_element_type=jnp.float32)
        bm = jnp.max(z, axis=1, keepdims=True)
        mz_old = m_ref[...]
        mz_new = jnp.maximum(mz_old, bm)
        e = jnp.exp2(_A * (z - mz_new))
        s_ref[...] = s_ref[...] * jnp.exp2(_A * (mz_old - mz_new)) + (
            jax.lax.dot(e, ones, preferred_element_type=jnp.float32))
        m_ref[...] = mz_new

    @pl.when(k < nk - 1)
    def _main():
        _accum(False)

    @pl.when(k == nk - 1)
    def _last():
        _accum(True)

        g = g_ref[...]
        spec2 = jnp.where(g < 1.0 + _M,
                          (2.0 * _A) * g - g * g * _A - 0.9375 * _A, 0.0)
        m2 = _A * m_ref[...] - (_A / 16.0)
        s_true = s_ref[...] + jnp.exp2(spec2 - m2)
        loss = (m2 + jnp.log2(s_true) - spec2) * _LN2
        out_ref[0, 0] = jnp.sum(loss) / loss.shape[0]


def kernel(inp, label):
    b, c = inp.shape
    block_cols = 2048
    nk = pl.cdiv(c, block_cols)
    lab2 = label.astype(jnp.int32).reshape(b, 1)
    out = pl.pallas_call(
        functools.partial(_loss_kernel, n_cols=c, block_cols=block_cols),
        grid=(nk,),
        in_specs=[
            pl.BlockSpec((b, 1), lambda k: (0, 0)),
            pl.BlockSpec((b, block_cols), lambda k: (0, k)),
        ],
        out_specs=pl.BlockSpec(memory_space=pltpu.SMEM),
        out_shape=jax.ShapeDtypeStruct((1, 1), jnp.float32),
        scratch_shapes=[
            pltpu.VMEM((b, 1), jnp.float32),
            pltpu.VMEM((b, 1), jnp.float32),
            pltpu.VMEM((b, 1), jnp.float32),
        ],
    )(lab2, inp)
    return out[0, 0]
